# baseline (speedup 1.0000x reference)
import functools

import jax
import jax.numpy as jnp
from jax import lax
from jax.experimental import pallas as pl
from jax.experimental.pallas import tpu as pltpu
from jax.experimental.pallas import tpu_sc as plsc

BATCH = 16384
HIST = 200
EMBED_DIM = 32
VOCAB = 1000000

_B = BATCH * HIST
_NC, _NS = 2, 16
_NW = _NC * _NS
_IW = 128
_UNITS = _B // _IW
_U_PER_W = _UNITS // _NW
_K = 8
_CHUNK = _IW * _K
_N_CHUNKS = _U_PER_W // _K
_HT = HIST // 8
_BT = BATCH // 128
_CT = EMBED_DIM // 8
_NTR = 4


def _emb_kernel(idx_hbm, tab_hbm, out_hbm, idx_v, rows_v, tr_v, idx_sems,
                g_sems, wb_sems):
    wid = lax.axis_index("s") * _NC + lax.axis_index("c")
    r0 = wid * _U_PER_W

    def idx_copy(c):
        buf = lax.rem(c, 3)
        return pltpu.make_async_copy(
            idx_hbm.at[pl.ds(r0 + c * _K, _K)],
            idx_v.at[buf],
            idx_sems.at[buf],
        )

    def gather(c, j):
        buf = lax.rem(c, 2)
        ibuf = lax.rem(c, 3)
        return pltpu.make_async_copy(
            tab_hbm.at[idx_v.at[ibuf, j]],
            rows_v.at[buf, pl.ds(j * _IW, _IW)],
            g_sems.at[buf],
        )

    def writeback(g, t, ct):
        ht = lax.div(g, 1024)
        bt = lax.rem(lax.div(g, 8), 128)
        hs = lax.rem(g, 8)
        return pltpu.make_async_copy(
            tr_v.at[t, pl.ds(ct * 8, 8)],
            out_hbm.at[ht * 8 + hs, ct, bt],
            wb_sems.at[t],
        )

    def fire_gathers(c):
        idx_copy(c).wait()
        for j in range(_K):
            gather(c, j).start()

    iota = lax.iota(jnp.int32, 16)
    rowvecs = [[iota + (u * _IW + l * 16) for l in range(8)] for u in range(_K)]
    colvecs = [jnp.full((16,), cc, jnp.int32) for cc in range(EMBED_DIM)]

    idx_copy(0).start()
    idx_copy(1).start()
    fire_gathers(0)

    @pl.loop(0, _N_CHUNKS)
    def _chunk(c):
        buf = lax.rem(c, 2)

        @pl.when(c + 2 < _N_CHUNKS)
        def _():
            idx_copy(c + 2).start()

        @pl.when(c + 1 < _N_CHUNKS)
        def _():
            fire_gathers(c + 1)

        for j in range(_K):
            gather(c, j).wait()

        for u in range(_K):
            g = r0 + c * _K + u
            t = u % _NTR

            @pl.when(c * _K + u >= _NTR)
            def _():
                for ct in range(_CT):
                    writeback(g - _NTR, t, ct).wait()

            for cc in range(EMBED_DIM):
                for l in range(8):
                    v = plsc.load_gather(
                        rows_v.at[buf], [rowvecs[u][l], colvecs[cc]]
                    )
                    tr_v[t, cc, pl.ds(l * 16, 16)] = v

            for ct in range(_CT):
                writeback(g, t, ct).start()

    last = r0 + _U_PER_W
    for d in range(_NTR):
        for ct in range(_CT):
            writeback(last - _NTR + d, d, ct).wait()


def kernel(inputs, table):
    idx = (
        inputs.T.reshape(_HT, 8, _BT, 128)
        .transpose(0, 2, 1, 3)
        .reshape(_UNITS, _IW)
    )
    mesh = plsc.VectorSubcoreMesh(core_axis_name="c", subcore_axis_name="s")
    run = functools.partial(
        pl.kernel,
        out_type=jax.ShapeDtypeStruct((HIST, _CT, _BT, 8, 128), jnp.float32),
        mesh=mesh,
        scratch_types=[
            pltpu.VMEM((3, _K, _IW), jnp.int32),
            pltpu.VMEM((2, _CHUNK, EMBED_DIM), jnp.float32),
            pltpu.VMEM((_NTR, EMBED_DIM, 128), jnp.float32),
            pltpu.SemaphoreType.DMA((3,)),
            pltpu.SemaphoreType.DMA((2,)),
            pltpu.SemaphoreType.DMA((_NTR,)),
        ],
        compiler_params=pltpu.CompilerParams(
            use_tc_tiling_on_sc=False, needs_layout_passes=False
        ),
    )(_emb_kernel)
    out = run(idx, table)
    return out.transpose(2, 4, 0, 1, 3).reshape(BATCH, HIST, EMBED_DIM)


# speedup vs baseline: 1.7353x; 1.7353x over previous
import functools

import jax
import jax.numpy as jnp
from jax import lax
from jax.experimental import pallas as pl
from jax.experimental.pallas import tpu as pltpu
from jax.experimental.pallas import tpu_sc as plsc

BATCH = 16384
HIST = 200
EMBED_DIM = 32
VOCAB = 1000000

_B = BATCH * HIST
_NC, _NS = 2, 16
_NW = _NC * _NS
_IW = 128
_UNITS = _B // _IW
_U_PER_W = _UNITS // _NW
_K = 8
_CHUNK = _IW * _K
_N_CHUNKS = _U_PER_W // _K
_HT = HIST // 8
_BT = BATCH // 128
_CT = EMBED_DIM // 8
_NTR = 4
_RP = 129


def _emb_kernel(idx_hbm, tab_hbm, out_hbm, idx_v, rows_v, tr_v, idx_sems,
                g_sems, wb_sems):
    wid = lax.axis_index("s") * _NC + lax.axis_index("c")
    r0 = wid * _U_PER_W

    def idx_copy(c):
        buf = lax.rem(c, 3)
        return pltpu.make_async_copy(
            idx_hbm.at[pl.ds(r0 + c * _K, _K)],
            idx_v.at[buf],
            idx_sems.at[buf],
        )

    def gather(c, j):
        buf = lax.rem(c, 2)
        ibuf = lax.rem(c, 3)
        return pltpu.make_async_copy(
            tab_hbm.at[idx_v.at[ibuf, j]],
            rows_v.at[buf, pl.ds(j * _IW, _IW)],
            g_sems.at[buf],
        )

    def writeback(g, t, ct):
        ht = lax.div(g, 1024)
        bt = lax.rem(lax.div(g, 8), 128)
        hs = lax.rem(g, 8)
        return pltpu.make_async_copy(
            tr_v.at[t, pl.ds(ct * 8, 8), pl.ds(0, 128)],
            out_hbm.at[ht * 8 + hs, ct, bt],
            wb_sems.at[t],
        )

    def fire_gathers(c):
        idx_copy(c).wait()
        for j in range(_K):
            gather(c, j).start()

    iota = lax.iota(jnp.int32, 16)

    idx_copy(0).start()
    idx_copy(1).start()
    fire_gathers(0)

    @pl.loop(0, _N_CHUNKS)
    def _chunk(c):
        buf = lax.rem(c, 2)

        @pl.when(c + 2 < _N_CHUNKS)
        def _():
            idx_copy(c + 2).start()

        @pl.when(c + 1 < _N_CHUNKS)
        def _():
            fire_gathers(c + 1)

        for j in range(_K):
            gather(c, j).wait()

        for u in range(_K):
            g = r0 + c * _K + u
            t = u % _NTR

            @pl.when(c * _K + u >= _NTR)
            def _():
                for ct in range(_CT):
                    writeback(g - _NTR, t, ct).wait()

            for l in range(_IW):
                lrow = jnp.full((16,), l, jnp.int32)
                for h in range(2):
                    v = rows_v[buf, u * _IW + l, pl.ds(h * 16, 16)]
                    plsc.store_scatter(
                        tr_v.at[t], [iota + h * 16, lrow], v
                    )

            for ct in range(_CT):
                writeback(g, t, ct).start()

    last = r0 + _U_PER_W
    for d in range(_NTR):
        for ct in range(_CT):
            writeback(last - _NTR + d, d, ct).wait()


def kernel(inputs, table):
    idx = (
        inputs.T.reshape(_HT, 8, _BT, 128)
        .transpose(0, 2, 1, 3)
        .reshape(_UNITS, _IW)
    )
    mesh = plsc.VectorSubcoreMesh(core_axis_name="c", subcore_axis_name="s")
    run = functools.partial(
        pl.kernel,
        out_type=jax.ShapeDtypeStruct((HIST, _CT, _BT, 8, 128), jnp.float32),
        mesh=mesh,
        scratch_types=[
            pltpu.VMEM((3, _K, _IW), jnp.int32),
            pltpu.VMEM((2, _CHUNK, EMBED_DIM), jnp.float32),
            pltpu.VMEM((_NTR, EMBED_DIM, _RP), jnp.float32),
            pltpu.SemaphoreType.DMA((3,)),
            pltpu.SemaphoreType.DMA((2,)),
            pltpu.SemaphoreType.DMA((_NTR,)),
        ],
        compiler_params=pltpu.CompilerParams(
            use_tc_tiling_on_sc=False, needs_layout_passes=False
        ),
    )(_emb_kernel)
    out = run(idx, table)
    return out.transpose(2, 4, 0, 1, 3).reshape(BATCH, HIST, EMBED_DIM)
